# Initial kernel scaffold; baseline (speedup 1.0000x reference)
import jax
import jax.numpy as jnp
from jax.experimental import pallas as pl


def kernel(x, major_table, accessory_table, W, b):
    raise NotImplementedError("write your pallas kernel here")


# speedup vs baseline: 1.1171x; 1.1171x over previous
import numpy as np
import jax
import jax.numpy as jnp
from jax import lax
from jax.experimental import pallas as pl
from jax.experimental.pallas import tpu as pltpu

_MAJOR = np.array([30, 32, 33, 46, 47, 48, 50, 54, 76, 82, 84, 88, 90], dtype=np.int32)
_ACC = np.array([10, 11, 16, 20, 24, 35, 36, 53, 62, 63, 71, 73, 74, 77, 85, 93], dtype=np.int32)
_P = 99
_E = 128
_NPOS = len(_MAJOR) + len(_ACC)
_NSEL = 32


def _sel_matrix() -> np.ndarray:
    sel = np.zeros((_P * 22, _NSEL), np.float32)
    for j, pos in enumerate(np.concatenate([_MAJOR, _ACC])):
        q = int(pos) - 1
        sel[22 * q: 22 * q + 21, j] = 1.0
    return sel


_SEL = _sel_matrix()


def _body(x_ref, sel_ref, mt_ref, at_ref, w_ref, b_ref, out_ref):
    xb = x_ref[...].astype(jnp.bfloat16)
    sums = lax.dot_general(xb, sel_ref[...], (((1,), (0,)), ((), ())),
                           preferred_element_type=jnp.float32)
    mut = (sums > 0).astype(jnp.float32)
    pm = lax.dot_general(mt_ref[...], w_ref[:, :_E], (((1,), (1,)), ((), ())),
                         preferred_element_type=jnp.float32)
    pa = lax.dot_general(at_ref[...], w_ref[:, _E:], (((1,), (1,)), ((), ())),
                         preferred_element_type=jnp.float32)
    proj = jnp.concatenate([pm, pa, jnp.zeros((_NSEL - _NPOS, _E), jnp.float32)], axis=0)
    out_ref[...] = lax.dot_general(mut, proj, (((1,), (0,)), ((), ())),
                                   preferred_element_type=jnp.float32) + b_ref[...]


def kernel(x, major_table, accessory_table, W, b):
    batch, feat = x.shape
    bb = 512
    grid = (batch // bb,)
    sel = jnp.asarray(_SEL, dtype=jnp.bfloat16)
    b2 = b.reshape(1, _E)
    return pl.pallas_call(
        _body,
        grid=grid,
        in_specs=[
            pl.BlockSpec((bb, feat), lambda i: (i, 0)),
            pl.BlockSpec((feat, _NSEL), lambda i: (0, 0)),
            pl.BlockSpec(major_table.shape, lambda i: (0, 0)),
            pl.BlockSpec(accessory_table.shape, lambda i: (0, 0)),
            pl.BlockSpec(W.shape, lambda i: (0, 0)),
            pl.BlockSpec((1, _E), lambda i: (0, 0)),
        ],
        out_specs=pl.BlockSpec((bb, _E), lambda i: (i, 0)),
        out_shape=jax.ShapeDtypeStruct((batch, _E), jnp.float32),
        compiler_params=pltpu.CompilerParams(
            dimension_semantics=("parallel",),
        ),
    )(x, sel, major_table, accessory_table, W, b2)
